# baseline (speedup 1.0000x reference)
import functools

import jax
import jax.numpy as jnp
from jax import lax
from jax.experimental import pallas as pl
from jax.experimental.pallas import tpu as pltpu
from jax.experimental.pallas import tpu_sc as plsc

B, K, H, W = 32, 17, 128, 128
HW = H * W
_FI = tuple(0 if k == 0 else (k + 1 if k % 2 == 1 else k - 1) for k in range(K))

_CHUNKS_PER_ROW = W // 16


def _swap_rev_pair(buf, negate):
    def body(h, carry):
        base = h * W
        for j in range(_CHUNKS_PER_ROW // 2):
            c1 = base + j * 16
            c2 = base + (_CHUNKS_PER_ROW - 1 - j) * 16
            va1 = lax.rev(buf[0, pl.ds(c1, 16)], (0,))
            va2 = lax.rev(buf[0, pl.ds(c2, 16)], (0,))
            vb1 = lax.rev(buf[1, pl.ds(c1, 16)], (0,))
            vb2 = lax.rev(buf[1, pl.ds(c2, 16)], (0,))
            if negate:
                va1, va2, vb1, vb2 = -va1, -va2, -vb1, -vb2
            buf[0, pl.ds(c1, 16)] = vb2
            buf[0, pl.ds(c2, 16)] = vb1
            buf[1, pl.ds(c1, 16)] = va2
            buf[1, pl.ds(c2, 16)] = va1
        return carry

    lax.fori_loop(0, H, body, 0)


def _self_rev(buf, negate):
    def body(h, carry):
        base = h * W
        for j in range(_CHUNKS_PER_ROW // 2):
            c1 = base + j * 16
            c2 = base + (_CHUNKS_PER_ROW - 1 - j) * 16
            v1 = lax.rev(buf[0, pl.ds(c1, 16)], (0,))
            v2 = lax.rev(buf[0, pl.ds(c2, 16)], (0,))
            if negate:
                v1, v2 = -v1, -v2
            buf[0, pl.ds(c1, 16)] = v2
            buf[0, pl.ds(c2, 16)] = v1
        return carry

    lax.fori_loop(0, H, body, 0)


def _sc_flip(conf, reg):
    mesh = plsc.VectorSubcoreMesh(core_axis_name="c", subcore_axis_name="s")

    @functools.partial(
        pl.kernel,
        mesh=mesh,
        out_type=(
            jax.ShapeDtypeStruct((B, K, HW), jnp.float32),
            jax.ShapeDtypeStruct((B, K, 2, HW), jnp.float32),
        ),
        scratch_types=[
            pltpu.VMEM((2, HW), jnp.float32),
            pltpu.VMEM((2, HW), jnp.float32),
            pltpu.VMEM((2, HW), jnp.float32),
            pltpu.SemaphoreType.DMA,
            pltpu.SemaphoreType.DMA,
            pltpu.SemaphoreType.DMA,
            pltpu.SemaphoreType.DMA,
            pltpu.SemaphoreType.DMA,
            pltpu.SemaphoreType.DMA,
        ],
        compiler_params=pltpu.CompilerParams(use_tc_tiling_on_sc=False),
    )
    def k(conf_in, reg_in, conf_out, reg_out,
          buf0, buf1, buf2, isem0, isem1, isem2, osem0, osem1, osem2):
        w = lax.axis_index("s") * 2 + lax.axis_index("c")
        bufs = (buf0, buf1, buf2)
        isems, osems = (isem0, isem1, isem2), (osem0, osem1, osem2)

        units = []
        units.append((conf_in.at[w, 0], conf_out.at[w, 0], False, False))
        for p in range(K // 2):
            sl = pl.ds(2 * p + 1, 2)
            units.append((conf_in.at[w, sl], conf_out.at[w, sl], True, False))
        for c in range(2):
            neg = c == 0
            units.append(
                (reg_in.at[w, 0, c], reg_out.at[w, 0, c], False, neg))
            for p in range(K // 2):
                sl = pl.ds(2 * p + 1, 2)
                units.append(
                    (reg_in.at[w, sl, c], reg_out.at[w, sl, c], True, neg))
        n = len(units)

        def start_in(i):
            src = units[i][0]
            dst = bufs[i % 3] if units[i][2] else bufs[i % 3].at[0]
            return pltpu.async_copy(src, dst, isems[i % 3])

        def start_out(i):
            src = bufs[i % 3] if units[i][2] else bufs[i % 3].at[0]
            return pltpu.async_copy(src, units[i][1], osems[i % 3])

        copy_in = [None] * n
        copy_out = [None] * n
        for i in range(3):
            copy_in[i] = start_in(i)
        for i in range(n):
            copy_in[i].wait()
            if i >= 2:
                copy_out[i - 2].wait()
                if i + 1 < n and copy_in[i + 1] is None:
                    copy_in[i + 1] = start_in(i + 1)
            if units[i][2]:
                _swap_rev_pair(bufs[i % 3], units[i][3])
            else:
                _self_rev(bufs[i % 3], units[i][3])
            copy_out[i] = start_out(i)
        copy_out[n - 2].wait()
        copy_out[n - 1].wait()

    return k(conf, reg)


def _fk(k):
    return jnp.where(k == 0, 0, jnp.where(k % 2 == 1, k + 1, k - 1))


def _tc_flip(x, bb=8):
    def body(in_ref, out_ref):
        r = lax.broadcasted_iota(jnp.int32, (W, W), 0)
        c = lax.broadcasted_iota(jnp.int32, (W, W), 1)
        j = jnp.where(r + c == W - 1, 1.0, 0.0).astype(jnp.float32)
        for i in range(bb):
            out_ref[i, 0] = jax.lax.dot(
                in_ref[i, 0], j, preferred_element_type=jnp.float32)

    return pl.pallas_call(
        body,
        grid=(B // bb, K),
        in_specs=[pl.BlockSpec((bb, 1, H, W), lambda b, k: (b, _fk(k), 0, 0))],
        out_specs=pl.BlockSpec((bb, 1, H, W), lambda b, k: (b, k, 0, 0)),
        out_shape=jax.ShapeDtypeStruct((B, K, H, W), jnp.float32),
    )(x)


def kernel(field_conf, field_reg, field_scale):
    conf = field_conf.reshape(B, K, HW)
    reg = field_reg.reshape(B, K, 2, HW)
    oc, orr = _sc_flip(conf, reg)
    osc = _tc_flip(field_scale)
    return (
        oc.reshape(B, K, H, W),
        orr.reshape(B, K, 2, H, W),
        osc,
    )
